# Initial kernel scaffold; baseline (speedup 1.0000x reference)
import jax
import jax.numpy as jnp
from jax.experimental import pallas as pl


def kernel(z, W_down, b_down, emb):
    raise NotImplementedError("write your pallas kernel here")


# speedup vs baseline: 1.0916x; 1.0916x over previous
import functools

import jax
import jax.numpy as jnp
from jax.experimental import pallas as pl
from jax.experimental.pallas import tpu as pltpu
from jax.experimental.pallas import tpu_sc as plsc

N = 16384
D = 1024
C = 32
K = 8192
HALF = K // 2
BLK = 512
GW = 128


def _first_argmin(dist, base):
    m = jnp.min(dist, axis=1)
    ii = jax.lax.broadcasted_iota(jnp.int32, dist.shape, 1)
    a = jnp.min(jnp.where(dist == m[:, None], ii, K), axis=1)
    return m, a + base


def _argmin_body(z_ref, w_ref, b_ref, emb_ref, zc2_ref, zc_aux_ref, idx_ref):
    del zc_aux_ref
    z = z_ref[...]
    w = w_ref[...]
    emb = emb_ref[...]
    dn = (((1,), (1,)), ((), ()))
    zc = jax.lax.dot_general(z, w, dn) + b_ref[...]
    mm = jax.lax.dot_general(zc, emb, dn)
    e2 = jnp.sum(emb ** 2, axis=1)[None, :]
    dist = zc2_ref[...] - 2.0 * mm + e2
    m0, a0 = _first_argmin(dist[:, :HALF], 0)
    m1, a1 = _first_argmin(dist[:, HALF:], HALF)
    m0b = m0.astype(jnp.bfloat16).astype(jnp.float32)
    idx_ref[0, 0, :] = jnp.where(m0b <= m1, a0, a1)


def _argmin_call(z, w, b, emb, zc2, zc_aux):
    return pl.pallas_call(
        _argmin_body,
        grid=(N // BLK,),
        in_specs=[
            pl.BlockSpec((BLK, D), lambda i: (i, 0)),
            pl.BlockSpec((C, D), lambda i: (0, 0)),
            pl.BlockSpec((1, C), lambda i: (0, 0)),
            pl.BlockSpec((K, C), lambda i: (0, 0)),
            pl.BlockSpec((BLK, 1), lambda i: (i, 0)),
            pl.BlockSpec((BLK, C), lambda i: (i, 0)),
        ],
        out_specs=pl.BlockSpec((1, 1, BLK), lambda i: (i, 0, 0)),
        out_shape=jax.ShapeDtypeStruct((N // BLK, 1, BLK), jnp.int32),
    )(z, w, b, emb, zc2, zc_aux)


def _gather_call(emb_pad, idx):
    idx2 = idx.reshape(1, N)
    mesh = plsc.VectorSubcoreMesh(core_axis_name="core",
                                  subcore_axis_name="subcore")

    @functools.partial(
        pl.kernel,
        out_type=jax.ShapeDtypeStruct((N, 128), jnp.float32),
        mesh=mesh,
    )
    def k(emb_hbm, i_hbm, o_hbm):
        def body(i_vmem, o_vmem):
            pltpu.sync_copy(emb_hbm.at[i_vmem.at[0]], o_vmem)

        pltpu.emit_pipeline(
            body,
            grid=(N // GW,),
            in_specs=[pl.BlockSpec((1, GW), index_map=lambda i: (0, i))],
            out_specs=[pl.BlockSpec((GW, 128), index_map=lambda i: (i, 0))],
            core_axis_name=("core", "subcore"),
            dimension_semantics=(pltpu.PARALLEL,),
        )(i_hbm, o_hbm)

    return k(emb_pad, idx2)


def kernel(z, W_down, b_down, emb):
    zc_aux = z @ W_down.T + b_down
    zc2 = jnp.sum(zc_aux ** 2, axis=1, keepdims=True)
    idx = _argmin_call(z, W_down, b_down.reshape(1, C), emb, zc2,
                       zc_aux).reshape(N)
    emb_pad = jnp.pad(emb, ((0, 0), (0, 128 - C)))
    z_st = _gather_call(emb_pad, idx)[:, :C]
    return (z_st, idx)


# speedup vs baseline: 1.1291x; 1.0344x over previous
import functools

import jax
import jax.numpy as jnp
from jax.experimental import pallas as pl
from jax.experimental.pallas import tpu as pltpu
from jax.experimental.pallas import tpu_sc as plsc

N = 16384
D = 1024
C = 32
K = 8192
HALF = K // 2
BLK = 512
GW = 128


def _first_argmin(dist, base):
    m = jnp.min(dist, axis=1)
    ii = jax.lax.broadcasted_iota(jnp.int32, dist.shape, 1)
    a = jnp.min(jnp.where(dist == m[:, None], ii, K), axis=1)
    return m, a + base


def _argmin_body(z_ref, w_ref, b_ref, emb_ref, zc2_ref, e2_ref, zc_aux_ref,
                 idx_ref):
    del zc_aux_ref
    z = z_ref[...]
    w = w_ref[...]
    emb = emb_ref[...]
    dn = (((1,), (1,)), ((), ()))
    zc = jax.lax.dot_general(z, w, dn) + b_ref[...]
    mm2 = jax.lax.dot_general(zc, emb * -2.0, dn)
    dist = zc2_ref[...] + mm2 + e2_ref[...]
    m0, a0 = _first_argmin(dist[:, :HALF], 0)
    m1, a1 = _first_argmin(dist[:, HALF:], HALF)
    m0b = m0.astype(jnp.bfloat16).astype(jnp.float32)
    idx_ref[0, 0, :] = jnp.where(m0b <= m1, a0, a1)


def _argmin_call(z, w, b, emb, zc2, e2, zc_aux):
    return pl.pallas_call(
        _argmin_body,
        grid=(N // BLK,),
        in_specs=[
            pl.BlockSpec((BLK, D), lambda i: (i, 0)),
            pl.BlockSpec((C, D), lambda i: (0, 0)),
            pl.BlockSpec((1, C), lambda i: (0, 0)),
            pl.BlockSpec((K, C), lambda i: (0, 0)),
            pl.BlockSpec((BLK, 1), lambda i: (i, 0)),
            pl.BlockSpec((1, K), lambda i: (0, 0)),
            pl.BlockSpec((BLK, C), lambda i: (i, 0)),
        ],
        out_specs=pl.BlockSpec((1, 1, BLK), lambda i: (i, 0, 0)),
        out_shape=jax.ShapeDtypeStruct((N // BLK, 1, BLK), jnp.int32),
    )(z, w, b, emb, zc2, e2, zc_aux)


def _gather_call(emb_pad, idx):
    idx2 = idx.reshape(1, N)
    mesh = plsc.VectorSubcoreMesh(core_axis_name="core",
                                  subcore_axis_name="subcore")

    @functools.partial(
        pl.kernel,
        out_type=jax.ShapeDtypeStruct((N, 128), jnp.float32),
        mesh=mesh,
    )
    def k(emb_hbm, i_hbm, o_hbm):
        def body(i_vmem, o_vmem):
            pltpu.sync_copy(emb_hbm.at[i_vmem.at[0]], o_vmem)

        pltpu.emit_pipeline(
            body,
            grid=(N // GW,),
            in_specs=[pl.BlockSpec((1, GW), index_map=lambda i: (0, i))],
            out_specs=[pl.BlockSpec((GW, 128), index_map=lambda i: (i, 0))],
            core_axis_name=("core", "subcore"),
            dimension_semantics=(pltpu.PARALLEL,),
        )(i_hbm, o_hbm)

    return k(emb_pad, idx2)


def kernel(z, W_down, b_down, emb):
    zc_aux = z @ W_down.T + b_down
    zc2 = jnp.sum(zc_aux ** 2, axis=1, keepdims=True)
    e2 = jnp.sum(emb ** 2, axis=1).reshape(1, K)
    idx = _argmin_call(z, W_down, b_down.reshape(1, C), emb, zc2, e2,
                       zc_aux).reshape(N)
    emb_pad = jnp.pad(emb, ((0, 0), (0, 128 - C)))
    z_st = _gather_call(emb_pad, idx)[:, :C]
    return (z_st, idx)


# speedup vs baseline: 1.2211x; 1.0815x over previous
import functools

import jax
import jax.numpy as jnp
from jax.experimental import pallas as pl
from jax.experimental.pallas import tpu as pltpu
from jax.experimental.pallas import tpu_sc as plsc

N = 16384
D = 1024
C = 32
K = 8192
HALF = K // 2
BLK = 512
GW = 128


def _first_argmin(dist, ii):
    m = jnp.min(dist, axis=1)
    a = jnp.min(jnp.where(dist == m[:, None], ii, float(K)), axis=1)
    return m, a


def _argmin_body(z_ref, w_ref, b_ref, emb2_ref, zc2_ref, e2_ref, ii_ref,
                 zc_aux_ref, idx_ref):
    del zc_aux_ref
    z = z_ref[...]
    w = w_ref[...]
    emb2 = emb2_ref[...]
    dn = (((1,), (1,)), ((), ()))
    zc = jax.lax.dot_general(z, w, dn) + b_ref[...]
    mm2 = jax.lax.dot_general(zc, emb2, dn)
    dist = zc2_ref[...] + mm2 + e2_ref[...]
    ii = ii_ref[...]
    m0, a0 = _first_argmin(dist[:, :HALF], ii)
    m1, a1 = _first_argmin(dist[:, HALF:], ii)
    m0b = m0.astype(jnp.bfloat16).astype(jnp.float32)
    idx = jnp.where(m0b <= m1, a0, a1 + float(HALF))
    idx_ref[0, 0, :] = idx.astype(jnp.int32)


def _argmin_call(z, w, b, emb2, zc2, e2, ii, zc_aux):
    return pl.pallas_call(
        _argmin_body,
        grid=(N // BLK,),
        in_specs=[
            pl.BlockSpec((BLK, D), lambda i: (i, 0)),
            pl.BlockSpec((C, D), lambda i: (0, 0)),
            pl.BlockSpec((1, C), lambda i: (0, 0)),
            pl.BlockSpec((K, C), lambda i: (0, 0)),
            pl.BlockSpec((BLK, 1), lambda i: (i, 0)),
            pl.BlockSpec((1, K), lambda i: (0, 0)),
            pl.BlockSpec((1, HALF), lambda i: (0, 0)),
            pl.BlockSpec((BLK, C), lambda i: (i, 0)),
        ],
        out_specs=pl.BlockSpec((1, 1, BLK), lambda i: (i, 0, 0)),
        out_shape=jax.ShapeDtypeStruct((N // BLK, 1, BLK), jnp.int32),
    )(z, w, b, emb2, zc2, e2, ii, zc_aux)


def _gather_call(emb_pad, idx):
    idx2 = idx.reshape(1, N)
    mesh = plsc.VectorSubcoreMesh(core_axis_name="core",
                                  subcore_axis_name="subcore")

    @functools.partial(
        pl.kernel,
        out_type=jax.ShapeDtypeStruct((N, 128), jnp.float32),
        mesh=mesh,
    )
    def k(emb_hbm, i_hbm, o_hbm):
        def body(i_vmem, o_vmem):
            pltpu.sync_copy(emb_hbm.at[i_vmem.at[0]], o_vmem)

        pltpu.emit_pipeline(
            body,
            grid=(N // GW,),
            in_specs=[pl.BlockSpec((1, GW), index_map=lambda i: (0, i))],
            out_specs=[pl.BlockSpec((GW, 128), index_map=lambda i: (i, 0))],
            core_axis_name=("core", "subcore"),
            dimension_semantics=(pltpu.PARALLEL,),
        )(i_hbm, o_hbm)

    return k(emb_pad, idx2)


def kernel(z, W_down, b_down, emb):
    zc_aux = z @ W_down.T + b_down
    zc2 = jnp.sum(zc_aux ** 2, axis=1, keepdims=True)
    e2 = jnp.sum(emb ** 2, axis=1).reshape(1, K)
    ii = jnp.arange(HALF, dtype=jnp.float32).reshape(1, HALF)
    idx = _argmin_call(z, W_down, b_down.reshape(1, C), emb * -2.0, zc2, e2,
                       ii, zc_aux).reshape(N)
    emb_pad = jnp.pad(emb, ((0, 0), (0, 128 - C)))
    z_st = _gather_call(emb_pad, idx)[:, :C]
    return (z_st, idx)


# speedup vs baseline: 1.4123x; 1.1565x over previous
import functools

import jax
import jax.numpy as jnp
from jax.experimental import pallas as pl
from jax.experimental.pallas import tpu as pltpu
from jax.experimental.pallas import tpu_sc as plsc

N = 16384
D = 1024
C = 32
K = 8192
HALF = K // 2
BLK = 512
GW = 128


RT = 64
CH = 128


def _window_argmin(zc2t, mmt, e2, lane_f, col0):
    acc_v = zc2t + mmt[:, col0:col0 + CH] + e2[:, col0:col0 + CH]
    acc_c = jnp.zeros_like(acc_v)
    for c in range(1, HALF // CH):
        d = zc2t + mmt[:, col0 + c * CH:col0 + (c + 1) * CH] \
            + e2[:, col0 + c * CH:col0 + (c + 1) * CH]
        take = d < acc_v
        acc_v = jnp.where(take, d, acc_v)
        acc_c = jnp.where(take, float(c), acc_c)
    m = jnp.min(acc_v, axis=1)
    jarr = acc_c * float(CH) + lane_f
    a = jnp.min(jnp.where(acc_v == m[:, None], jarr, float(K)), axis=1)
    return m, a


def _argmin_body(z_ref, w_ref, b_ref, emb2_ref, zc2_ref, e2_ref,
                 zc_aux_ref, idx_ref):
    del zc_aux_ref
    z = z_ref[...]
    w = w_ref[...]
    emb2 = emb2_ref[...]
    dn = (((1,), (1,)), ((), ()))
    zc = jax.lax.dot_general(z, w, dn) + b_ref[...]
    mm2 = jax.lax.dot_general(zc, emb2, dn)
    zc2 = zc2_ref[...]
    e2 = e2_ref[...]
    lane_f = jax.lax.broadcasted_iota(jnp.int32, (RT, CH), 1).astype(
        jnp.float32)
    for r in range(BLK // RT):
        zc2t = zc2[r * RT:(r + 1) * RT, :]
        mmt = mm2[r * RT:(r + 1) * RT, :]
        m0, a0 = _window_argmin(zc2t, mmt, e2, lane_f, 0)
        m1, a1 = _window_argmin(zc2t, mmt, e2, lane_f, HALF)
        m0b = m0.astype(jnp.bfloat16).astype(jnp.float32)
        idx = jnp.where(m0b <= m1, a0, a1 + float(HALF))
        idx_ref[0, 0, r * RT:(r + 1) * RT] = idx.astype(jnp.int32)


def _argmin_call(z, w, b, emb2, zc2, e2, zc_aux):
    return pl.pallas_call(
        _argmin_body,
        grid=(N // BLK,),
        in_specs=[
            pl.BlockSpec((BLK, D), lambda i: (i, 0)),
            pl.BlockSpec((C, D), lambda i: (0, 0)),
            pl.BlockSpec((1, C), lambda i: (0, 0)),
            pl.BlockSpec((K, C), lambda i: (0, 0)),
            pl.BlockSpec((BLK, 1), lambda i: (i, 0)),
            pl.BlockSpec((1, K), lambda i: (0, 0)),
            pl.BlockSpec((BLK, C), lambda i: (i, 0)),
        ],
        out_specs=pl.BlockSpec((1, 1, BLK), lambda i: (i, 0, 0)),
        out_shape=jax.ShapeDtypeStruct((N // BLK, 1, BLK), jnp.int32),
    )(z, w, b, emb2, zc2, e2, zc_aux)


def _gather_call(emb_pad, idx):
    idx2 = idx.reshape(1, N)
    mesh = plsc.VectorSubcoreMesh(core_axis_name="core",
                                  subcore_axis_name="subcore")

    @functools.partial(
        pl.kernel,
        out_type=jax.ShapeDtypeStruct((N, 128), jnp.float32),
        mesh=mesh,
    )
    def k(emb_hbm, i_hbm, o_hbm):
        def body(i_vmem, o_vmem):
            pltpu.sync_copy(emb_hbm.at[i_vmem.at[0]], o_vmem)

        pltpu.emit_pipeline(
            body,
            grid=(N // GW,),
            in_specs=[pl.BlockSpec((1, GW), index_map=lambda i: (0, i))],
            out_specs=[pl.BlockSpec((GW, 128), index_map=lambda i: (i, 0))],
            core_axis_name=("core", "subcore"),
            dimension_semantics=(pltpu.PARALLEL,),
        )(i_hbm, o_hbm)

    return k(emb_pad, idx2)


def kernel(z, W_down, b_down, emb):
    zc_aux = z @ W_down.T + b_down
    zc2 = jnp.sum(zc_aux ** 2, axis=1, keepdims=True)
    e2 = jnp.sum(emb ** 2, axis=1).reshape(1, K)
    idx = _argmin_call(z, W_down, b_down.reshape(1, C), emb * -2.0, zc2, e2,
                       zc_aux).reshape(N)
    emb_pad = jnp.pad(emb, ((0, 0), (0, 128 - C)))
    z_st = _gather_call(emb_pad, idx)[:, :C]
    return (z_st, idx)


# speedup vs baseline: 1.4402x; 1.0197x over previous
import functools

import jax
import jax.numpy as jnp
from jax.experimental import pallas as pl
from jax.experimental.pallas import tpu as pltpu
from jax.experimental.pallas import tpu_sc as plsc

N = 16384
D = 1024
C = 32
K = 8192
HALF = K // 2
BLK = 512
GW = 128


RT = 64
CH = 128


def _window_argmin(zc2t, mmt, e2, lane_f, col0):
    acc_v = zc2t + mmt[:, col0:col0 + CH] + e2[:, col0:col0 + CH]
    acc_c = jnp.zeros_like(acc_v)
    for c in range(1, HALF // CH):
        d = zc2t + mmt[:, col0 + c * CH:col0 + (c + 1) * CH] \
            + e2[:, col0 + c * CH:col0 + (c + 1) * CH]
        take = d < acc_v
        acc_v = jnp.where(take, d, acc_v)
        acc_c = jnp.where(take, float(c), acc_c)
    m = jnp.min(acc_v, axis=1)
    jarr = acc_c * float(CH) + lane_f
    a = jnp.min(jnp.where(acc_v == m[:, None], jarr, float(K)), axis=1)
    return m, a


def _zc_body(z_ref, w_ref, b_ref, zc_ref):
    dn = (((1,), (1,)), ((), ()))
    zc_ref[...] = jax.lax.dot_general(z_ref[...], w_ref[...], dn) + b_ref[...]


def _zc_call(z, w, b):
    return pl.pallas_call(
        _zc_body,
        grid=(N // BLK,),
        in_specs=[
            pl.BlockSpec((BLK, D), lambda i: (i, 0)),
            pl.BlockSpec((C, D), lambda i: (0, 0)),
            pl.BlockSpec((1, C), lambda i: (0, 0)),
        ],
        out_specs=pl.BlockSpec((BLK, C), lambda i: (i, 0)),
        out_shape=jax.ShapeDtypeStruct((N, C), jnp.float32),
    )(z, w, b)


def _argmin_body(zc_ref, emb2_ref, zc2_ref, e2_ref, idx_ref):
    zc = zc_ref[...]
    emb2 = emb2_ref[...]
    dn = (((1,), (1,)), ((), ()))
    mm2 = jax.lax.dot_general(zc, emb2, dn)
    zc2 = zc2_ref[...]
    e2 = e2_ref[...]
    lane_f = jax.lax.broadcasted_iota(jnp.int32, (RT, CH), 1).astype(
        jnp.float32)
    for r in range(BLK // RT):
        zc2t = zc2[r * RT:(r + 1) * RT, :]
        mmt = mm2[r * RT:(r + 1) * RT, :]
        m0, a0 = _window_argmin(zc2t, mmt, e2, lane_f, 0)
        m1, a1 = _window_argmin(zc2t, mmt, e2, lane_f, HALF)
        m0b = m0.astype(jnp.bfloat16).astype(jnp.float32)
        idx = jnp.where(m0b <= m1, a0, a1 + float(HALF))
        idx_ref[0, 0, r * RT:(r + 1) * RT] = idx.astype(jnp.int32)


def _argmin_call(zc, emb2, zc2, e2):
    return pl.pallas_call(
        _argmin_body,
        grid=(N // BLK,),
        in_specs=[
            pl.BlockSpec((BLK, C), lambda i: (i, 0)),
            pl.BlockSpec((K, C), lambda i: (0, 0)),
            pl.BlockSpec((BLK, 1), lambda i: (i, 0)),
            pl.BlockSpec((1, K), lambda i: (0, 0)),
        ],
        out_specs=pl.BlockSpec((1, 1, BLK), lambda i: (i, 0, 0)),
        out_shape=jax.ShapeDtypeStruct((N // BLK, 1, BLK), jnp.int32),
    )(zc, emb2, zc2, e2)


def _gather_call(emb_pad, idx):
    idx2 = idx.reshape(1, N)
    mesh = plsc.VectorSubcoreMesh(core_axis_name="core",
                                  subcore_axis_name="subcore")

    @functools.partial(
        pl.kernel,
        out_type=jax.ShapeDtypeStruct((N, 128), jnp.float32),
        mesh=mesh,
    )
    def k(emb_hbm, i_hbm, o_hbm):
        def body(i_vmem, o_vmem):
            pltpu.sync_copy(emb_hbm.at[i_vmem.at[0]], o_vmem)

        pltpu.emit_pipeline(
            body,
            grid=(N // GW,),
            in_specs=[pl.BlockSpec((1, GW), index_map=lambda i: (0, i))],
            out_specs=[pl.BlockSpec((GW, 128), index_map=lambda i: (i, 0))],
            core_axis_name=("core", "subcore"),
            dimension_semantics=(pltpu.PARALLEL,),
        )(i_hbm, o_hbm)

    return k(emb_pad, idx2)


def kernel(z, W_down, b_down, emb):
    zc = _zc_call(z, W_down, b_down.reshape(1, C))
    zc2 = jnp.sum(zc ** 2, axis=1, keepdims=True)
    e2 = jnp.sum(emb ** 2, axis=1).reshape(1, K)
    idx = _argmin_call(zc, emb * -2.0, zc2, e2).reshape(N)
    emb_pad = jnp.pad(emb, ((0, 0), (0, 128 - C)))
    z_st = _gather_call(emb_pad, idx)[:, :C]
    return (z_st, idx)
